# baseline (speedup 1.0000x reference)
import functools

import jax
import jax.numpy as jnp
from jax import lax
from jax.experimental import pallas as pl
from jax.experimental.pallas import tpu as pltpu
from jax.experimental.pallas import tpu_sc as plsc

B = 30000
N = 100000
D = 128
A = 64
K = 16

NW = 32
CHUNK = 120
NBUF = 5

TB = 1200


def _sc_gather(table, idx3):
  w = table.shape[1]
  nchunk = idx3.shape[1]
  ngroup = nchunk // NBUF
  rows_per_w = nchunk * CHUNK
  mesh = plsc.VectorSubcoreMesh(core_axis_name="c", subcore_axis_name="s")

  @functools.partial(
      pl.kernel,
      out_type=jax.ShapeDtypeStruct((NW * rows_per_w, w), table.dtype),
      mesh=mesh,
      scratch_types=[
          pltpu.VMEM((nchunk, CHUNK), jnp.int32),
          pltpu.VMEM((NBUF, CHUNK, w), table.dtype),
      ] + [pltpu.SemaphoreType.DMA] * (2 * NBUF),
  )
  def gather_kernel(table_hbm, idx_hbm, out_hbm, idx_v, rows, *sems):
    gsem = sems[:NBUF]
    wsem = sems[NBUF:]
    wid = lax.axis_index("s") * 2 + lax.axis_index("c")
    base = wid * rows_per_w
    pltpu.sync_copy(idx_hbm.at[wid], idx_v)
    for b in range(NBUF):
      pltpu.async_copy(table_hbm.at[idx_v.at[b]], rows.at[b], gsem[b])

    def wait_write(b):
      pltpu.make_async_copy(rows.at[b], out_hbm.at[pl.ds(base, CHUNK)],
                            wsem[b]).wait()

    def group(g, carry):
      for b in range(NBUF):
        c = g * NBUF + b
        b2 = (b - 1) % NBUF
        pltpu.make_async_copy(table_hbm.at[idx_v.at[0]], rows.at[b],
                              gsem[b]).wait()
        pltpu.async_copy(rows.at[b], out_hbm.at[pl.ds(base + c * CHUNK,
                                                      CHUNK)], wsem[b])
        @pl.when(c >= 1)
        def _():
          wait_write(b2)

          @pl.when(c + NBUF - 1 < nchunk)
          def _():
            pltpu.async_copy(table_hbm.at[idx_v.at[c + NBUF - 1]],
                             rows.at[b2], gsem[b2])
      return carry

    lax.fori_loop(0, ngroup, group, 0)
    wait_write(NBUF - 1)

  return gather_kernel(table, idx3)


def _tc_body(q_ref, nb_ref, w_ref, e_ref, wq_ref, wk_ref, wv_ref, gw_ref,
             gb_ref, g_ref, b_ref, out_ref):
  q = q_ref[...]
  w = w_ref[...]
  t = lax.dot_general(q, wq_ref[...], (((1,), (1,)), ((), ())),
                      preferred_element_type=jnp.float32)
  u = jnp.dot(t, wk_ref[...], preferred_element_type=jnp.float32)
  u = u * (A ** -0.5)
  scores = jnp.dot(nb_ref[0] * u, e_ref[0],
                   preferred_element_type=jnp.float32)
  for j in range(1, K):
    scores = scores + jnp.dot(nb_ref[j] * u, e_ref[j],
                              preferred_element_type=jnp.float32)
  s = jnp.sum(w, axis=-1, keepdims=True)
  scores = scores + jnp.log(w + 1e-6 * s + 1e-12)
  m = jnp.max(scores, axis=-1, keepdims=True)
  e = jnp.exp(scores - m)
  attn = e / jnp.sum(e, axis=-1, keepdims=True)
  acc = attn[:, 0:1] * nb_ref[0]
  for j in range(1, K):
    acc = acc + attn[:, j:j + 1] * nb_ref[j]
  ctx = lax.dot_general(acc, wv_ref[...], (((1,), (1,)), ((), ())),
                        preferred_element_type=jnp.float32)
  gate = jax.nn.sigmoid(jnp.sum(q * gw_ref[...], axis=-1, keepdims=True)
                        + gb_ref[0, 0])
  x = q + gate * ctx
  mu = jnp.mean(x, axis=-1, keepdims=True)
  var = jnp.mean((x - mu) ** 2, axis=-1, keepdims=True)
  out_ref[...] = (x - mu) * lax.rsqrt(var + 1e-5) * g_ref[...] + b_ref[...]


def _tc_math(query_emb, nb3, neighbor_weights, esel, Wq, Wk, Wv, gate_w,
             gate_b2, ln_g2, ln_b2):
  cb = query_emb.shape[0]
  grid = (cb // TB,)
  return pl.pallas_call(
      _tc_body,
      out_shape=jax.ShapeDtypeStruct((cb, D), jnp.float32),
      grid=grid,
      in_specs=[
          pl.BlockSpec((TB, D), lambda i: (i, 0)),
          pl.BlockSpec((K, TB, D), lambda i: (0, i, 0)),
          pl.BlockSpec((TB, K), lambda i: (i, 0)),
          pl.BlockSpec((K, D, K), lambda i: (0, 0, 0)),
          pl.BlockSpec((A, D), lambda i: (0, 0)),
          pl.BlockSpec((A, D), lambda i: (0, 0)),
          pl.BlockSpec((D, D), lambda i: (0, 0)),
          pl.BlockSpec((1, D), lambda i: (0, 0)),
          pl.BlockSpec((1, 1), lambda i: (0, 0)),
          pl.BlockSpec((1, D), lambda i: (0, 0)),
          pl.BlockSpec((1, D), lambda i: (0, 0)),
      ],
      out_specs=pl.BlockSpec((TB, D), lambda i: (i, 0)),
      compiler_params=pltpu.CompilerParams(
          dimension_semantics=("parallel",)),
  )(query_emb, nb3, neighbor_weights, esel, Wq, Wk, Wv, gate_w, gate_b2,
    ln_g2, ln_b2)


SPLITS = (9600, 9600, 10800)


def kernel(query_emb, all_emb, neighbor_indices, neighbor_weights, Wq, Wk, Wv,
           gate_w, gate_b, ln_g, ln_b):
  gb2 = gate_b.reshape(1, 1)
  lg2 = ln_g.reshape(1, D)
  lb2 = ln_b.reshape(1, D)
  idxT = neighbor_indices.T
  esel = jnp.zeros((K, D, K), jnp.float32).at[
      jnp.arange(K)[:, None], jnp.arange(D)[None, :],
      jnp.arange(K)[:, None]].set(1.0)
  outs = []
  b0 = 0
  for cb in SPLITS:
    idx3 = idxT[:, b0:b0 + cb].reshape(NW, cb * K // (NW * CHUNK), CHUNK)
    nb3 = _sc_gather(all_emb, idx3).reshape(K, cb, D)
    outs.append(_tc_math(query_emb[b0:b0 + cb], nb3,
                         neighbor_weights[b0:b0 + cb], esel, Wq, Wk, Wv,
                         gate_w, gb2, lg2, lb2))
    b0 += cb
  return jnp.concatenate(outs, axis=0)


# speedup vs baseline: 1.0290x; 1.0290x over previous
import functools

import jax
import jax.numpy as jnp
from jax import lax
from jax.experimental import pallas as pl
from jax.experimental.pallas import tpu as pltpu
from jax.experimental.pallas import tpu_sc as plsc

B = 30000
N = 100000
D = 128
A = 64
K = 16

NW = 32
CHUNK = 120
NBUF = 5

TB = 1200


def _sc_gather(table, idx3):
  w = table.shape[1]
  nchunk = idx3.shape[1]
  ngroup = nchunk // NBUF
  rows_per_w = nchunk * CHUNK
  mesh = plsc.VectorSubcoreMesh(core_axis_name="c", subcore_axis_name="s")

  @functools.partial(
      pl.kernel,
      out_type=jax.ShapeDtypeStruct((NW * rows_per_w, w), table.dtype),
      mesh=mesh,
      scratch_types=[
          pltpu.VMEM((nchunk, CHUNK), jnp.int32),
          pltpu.VMEM((NBUF, CHUNK, w), table.dtype),
      ] + [pltpu.SemaphoreType.DMA] * (2 * NBUF),
  )
  def gather_kernel(table_hbm, idx_hbm, out_hbm, idx_v, rows, *sems):
    gsem = sems[:NBUF]
    wsem = sems[NBUF:]
    wid = lax.axis_index("s") * 2 + lax.axis_index("c")
    base = wid * rows_per_w
    pltpu.sync_copy(idx_hbm.at[wid], idx_v)
    for b in range(NBUF):
      pltpu.async_copy(table_hbm.at[idx_v.at[b]], rows.at[b], gsem[b])

    def wait_write(b):
      pltpu.make_async_copy(rows.at[b], out_hbm.at[pl.ds(base, CHUNK)],
                            wsem[b]).wait()

    def group(g, carry):
      for b in range(NBUF):
        c = g * NBUF + b
        b2 = (b - 1) % NBUF
        pltpu.make_async_copy(table_hbm.at[idx_v.at[0]], rows.at[b],
                              gsem[b]).wait()
        pltpu.async_copy(rows.at[b], out_hbm.at[pl.ds(base + c * CHUNK,
                                                      CHUNK)], wsem[b])
        @pl.when(c >= 1)
        def _():
          wait_write(b2)

          @pl.when(c + NBUF - 1 < nchunk)
          def _():
            pltpu.async_copy(table_hbm.at[idx_v.at[c + NBUF - 1]],
                             rows.at[b2], gsem[b2])
      return carry

    lax.fori_loop(0, ngroup, group, 0)
    wait_write(NBUF - 1)

  return gather_kernel(table, idx3)


def _tc_body(q_ref, nb_ref, w_ref, e_ref, wq_ref, wk_ref, wv_ref, gw_ref,
             gb_ref, g_ref, b_ref, out_ref):
  q = q_ref[...]
  w = w_ref[...]
  t = lax.dot_general(q, wq_ref[...], (((1,), (1,)), ((), ())),
                      preferred_element_type=jnp.float32)
  u = jnp.dot(t, wk_ref[...], preferred_element_type=jnp.float32)
  u = u * (A ** -0.5)
  scores = jnp.dot(nb_ref[0] * u, e_ref[0],
                   preferred_element_type=jnp.float32)
  for j in range(1, K):
    scores = scores + jnp.dot(nb_ref[j] * u, e_ref[j],
                              preferred_element_type=jnp.float32)
  s = jnp.sum(w, axis=-1, keepdims=True)
  scores = scores + jnp.log(w + 1e-6 * s + 1e-12)
  m = jnp.max(scores, axis=-1, keepdims=True)
  e = jnp.exp(scores - m)
  attn = e / jnp.sum(e, axis=-1, keepdims=True)
  acc = attn[:, 0:1] * nb_ref[0]
  for j in range(1, K):
    acc = acc + attn[:, j:j + 1] * nb_ref[j]
  ctx = lax.dot_general(acc, wv_ref[...], (((1,), (1,)), ((), ())),
                        preferred_element_type=jnp.float32)
  gate = jax.nn.sigmoid(jnp.sum(q * gw_ref[...], axis=-1, keepdims=True)
                        + gb_ref[0, 0])
  x = q + gate * ctx
  mu = jnp.mean(x, axis=-1, keepdims=True)
  var = jnp.mean((x - mu) ** 2, axis=-1, keepdims=True)
  out_ref[...] = (x - mu) * lax.rsqrt(var + 1e-5) * g_ref[...] + b_ref[...]


def _tc_math(query_emb, nb3, neighbor_weights, esel, Wq, Wk, Wv, gate_w,
             gate_b2, ln_g2, ln_b2, cb, off):
  grid = (cb // TB,)
  return pl.pallas_call(
      _tc_body,
      out_shape=jax.ShapeDtypeStruct((cb, D), jnp.float32),
      grid=grid,
      in_specs=[
          pl.BlockSpec((TB, D), lambda i: (i + off, 0)),
          pl.BlockSpec((K, TB, D), lambda i: (0, i, 0)),
          pl.BlockSpec((TB, K), lambda i: (i + off, 0)),
          pl.BlockSpec((K, D, K), lambda i: (0, 0, 0)),
          pl.BlockSpec((A, D), lambda i: (0, 0)),
          pl.BlockSpec((A, D), lambda i: (0, 0)),
          pl.BlockSpec((D, D), lambda i: (0, 0)),
          pl.BlockSpec((1, D), lambda i: (0, 0)),
          pl.BlockSpec((1, 1), lambda i: (0, 0)),
          pl.BlockSpec((1, D), lambda i: (0, 0)),
          pl.BlockSpec((1, D), lambda i: (0, 0)),
      ],
      out_specs=pl.BlockSpec((TB, D), lambda i: (i, 0)),
      compiler_params=pltpu.CompilerParams(
          dimension_semantics=("parallel",)),
  )(query_emb, nb3, neighbor_weights, esel, Wq, Wk, Wv, gate_w, gate_b2,
    ln_g2, ln_b2)


SPLITS = (9600, 9600, 10800)


def kernel(query_emb, all_emb, neighbor_indices, neighbor_weights, Wq, Wk, Wv,
           gate_w, gate_b, ln_g, ln_b):
  gb2 = gate_b.reshape(1, 1)
  lg2 = ln_g.reshape(1, D)
  lb2 = ln_b.reshape(1, D)
  idxT = neighbor_indices.T
  esel = jnp.zeros((K, D, K), jnp.float32).at[
      jnp.arange(K)[:, None], jnp.arange(D)[None, :],
      jnp.arange(K)[:, None]].set(1.0)
  outs = []
  b0 = 0
  for cb in SPLITS:
    idx3 = idxT[:, b0:b0 + cb].reshape(NW, cb * K // (NW * CHUNK), CHUNK)
    nb3 = _sc_gather(all_emb, idx3).reshape(K, cb, D)
    outs.append(_tc_math(query_emb, nb3, neighbor_weights, esel, Wq, Wk, Wv,
                         gate_w, gb2, lg2, lb2, cb, b0 // TB))
    b0 += cb
  return jnp.concatenate(outs, axis=0)


# speedup vs baseline: 1.0821x; 1.0516x over previous
import functools

import jax
import jax.numpy as jnp
from jax import lax
from jax.experimental import pallas as pl
from jax.experimental.pallas import tpu as pltpu
from jax.experimental.pallas import tpu_sc as plsc

B = 30000
N = 100000
D = 128
A = 64
K = 16

NW = 32
CHUNK = 120
NBUF = 5

TB = 1200


def _sc_gather(table, idx3):
  w = table.shape[1]
  nchunk = idx3.shape[1]
  ngroup = nchunk // NBUF
  rows_per_w = nchunk * CHUNK
  mesh = plsc.VectorSubcoreMesh(core_axis_name="c", subcore_axis_name="s")

  @functools.partial(
      pl.kernel,
      out_type=jax.ShapeDtypeStruct((NW * rows_per_w, w), table.dtype),
      mesh=mesh,
      scratch_types=[
          pltpu.VMEM((nchunk, CHUNK), jnp.int32),
          pltpu.VMEM((NBUF, CHUNK, w), table.dtype),
      ] + [pltpu.SemaphoreType.DMA] * (2 * NBUF),
  )
  def gather_kernel(table_hbm, idx_hbm, out_hbm, idx_v, rows, *sems):
    gsem = sems[:NBUF]
    wsem = sems[NBUF:]
    wid = lax.axis_index("s") * 2 + lax.axis_index("c")
    base = wid * rows_per_w
    pltpu.sync_copy(idx_hbm.at[wid], idx_v)
    for b in range(NBUF):
      pltpu.async_copy(table_hbm.at[idx_v.at[b]], rows.at[b], gsem[b])

    def wait_write(b):
      pltpu.make_async_copy(rows.at[b], out_hbm.at[pl.ds(base, CHUNK)],
                            wsem[b]).wait()

    def group(g, carry):
      for b in range(NBUF):
        c = g * NBUF + b
        b2 = (b - 1) % NBUF
        pltpu.make_async_copy(table_hbm.at[idx_v.at[0]], rows.at[b],
                              gsem[b]).wait()
        pltpu.async_copy(rows.at[b], out_hbm.at[pl.ds(base + c * CHUNK,
                                                      CHUNK)], wsem[b])
        @pl.when(c >= 1)
        def _():
          wait_write(b2)

          @pl.when(c + NBUF - 1 < nchunk)
          def _():
            pltpu.async_copy(table_hbm.at[idx_v.at[c + NBUF - 1]],
                             rows.at[b2], gsem[b2])
      return carry

    lax.fori_loop(0, ngroup, group, 0)
    wait_write(NBUF - 1)

  return gather_kernel(table, idx3)


def _tc_body(q_ref, nb_ref, w_ref, e_ref, wq_ref, wk_ref, wv_ref, gw_ref,
             gb_ref, g_ref, b_ref, out_ref):
  q = q_ref[...]
  w = w_ref[...]
  t = lax.dot_general(q, wq_ref[...], (((1,), (1,)), ((), ())),
                      preferred_element_type=jnp.float32)
  u = jnp.dot(t, wk_ref[...], preferred_element_type=jnp.float32)
  u = u * (A ** -0.5)
  scores = jnp.dot(nb_ref[0] * u, e_ref[0],
                   preferred_element_type=jnp.float32)
  for j in range(1, K):
    scores = scores + jnp.dot(nb_ref[j] * u, e_ref[j],
                              preferred_element_type=jnp.float32)
  s = jnp.sum(w, axis=-1, keepdims=True)
  scores = scores + jnp.log(w + 1e-6 * s + 1e-12)
  m = jnp.max(scores, axis=-1, keepdims=True)
  e = jnp.exp(scores - m)
  attn = e / jnp.sum(e, axis=-1, keepdims=True)
  acc = attn[:, 0:1] * nb_ref[0]
  for j in range(1, K):
    acc = acc + attn[:, j:j + 1] * nb_ref[j]
  ctx = lax.dot_general(acc, wv_ref[...], (((1,), (1,)), ((), ())),
                        preferred_element_type=jnp.float32)
  gate = jax.nn.sigmoid(jnp.sum(q * gw_ref[...], axis=-1, keepdims=True)
                        + gb_ref[0, 0])
  x = q + gate * ctx
  mu = jnp.mean(x, axis=-1, keepdims=True)
  var = jnp.mean((x - mu) ** 2, axis=-1, keepdims=True)
  out_ref[...] = (x - mu) * lax.rsqrt(var + 1e-5) * g_ref[...] + b_ref[...]


def _tc_math(query_emb, nb3, neighbor_weights, esel, Wq, Wk, Wv, gate_w,
             gate_b2, ln_g2, ln_b2, cb, off, prev):
  grid = (cb // TB,)
  specs = [
      pl.BlockSpec((TB, D), lambda i: (i + off, 0)),
      pl.BlockSpec((K, TB, D), lambda i: (0, i, 0)),
      pl.BlockSpec((TB, K), lambda i: (i + off, 0)),
      pl.BlockSpec((K, D, K), lambda i: (0, 0, 0)),
      pl.BlockSpec((A, D), lambda i: (0, 0)),
      pl.BlockSpec((A, D), lambda i: (0, 0)),
      pl.BlockSpec((D, D), lambda i: (0, 0)),
      pl.BlockSpec((1, D), lambda i: (0, 0)),
      pl.BlockSpec((1, 1), lambda i: (0, 0)),
      pl.BlockSpec((1, D), lambda i: (0, 0)),
      pl.BlockSpec((1, D), lambda i: (0, 0)),
  ]
  args = [query_emb, nb3, neighbor_weights, esel, Wq, Wk, Wv, gate_w,
          gate_b2, ln_g2, ln_b2]
  if prev is None:
    body = _tc_body
    aliases = {}
  else:
    body = lambda prev_ref, *refs: _tc_body(*refs)
    specs = [pl.BlockSpec(memory_space=pl.ANY)] + specs
    args = [prev] + args
    aliases = {0: 0}
  return pl.pallas_call(
      body,
      out_shape=jax.ShapeDtypeStruct((B, D), jnp.float32),
      grid=grid,
      in_specs=specs,
      out_specs=pl.BlockSpec((TB, D), lambda i: (i + off, 0)),
      input_output_aliases=aliases,
      compiler_params=pltpu.CompilerParams(
          dimension_semantics=("parallel",)),
  )(*args)


SPLITS = (9600, 9600, 10800)


def kernel(query_emb, all_emb, neighbor_indices, neighbor_weights, Wq, Wk, Wv,
           gate_w, gate_b, ln_g, ln_b):
  gb2 = gate_b.reshape(1, 1)
  lg2 = ln_g.reshape(1, D)
  lb2 = ln_b.reshape(1, D)
  idxT = neighbor_indices.T
  esel = jnp.zeros((K, D, K), jnp.float32).at[
      jnp.arange(K)[:, None], jnp.arange(D)[None, :],
      jnp.arange(K)[:, None]].set(1.0)
  out = None
  b0 = 0
  for cb in SPLITS:
    idx3 = idxT[:, b0:b0 + cb].reshape(NW, cb * K // (NW * CHUNK), CHUNK)
    nb3 = _sc_gather(all_emb, idx3).reshape(K, cb, D)
    out = _tc_math(query_emb, nb3, neighbor_weights, esel, Wq, Wk, Wv,
                   gate_w, gb2, lg2, lb2, cb, b0 // TB, out)
    b0 += cb
  return out
